# baseline (speedup 1.0000x reference)
import jax
import jax.numpy as jnp
from jax import lax
from jax.experimental import pallas as pl
from jax.experimental.pallas import tpu as pltpu
from jax.experimental.pallas import tpu_sc as plsc
from jax.experimental.layout import Format, Layout

NUM_PRIM = 8192
DIM = 64
BATCH = 64
HW = 1024
N = BATCH * HW
NC, NS = 2, 16
NW = NC * NS
CHUNK = 128
PER_W = N // NW
NCH = PER_W // CHUNK
B_PER_W = PER_W // HW
CH_PER_B = HW // CHUNK
HCH = (N // NS) // CHUNK
CSLICE = NUM_PRIM // NS
LANES = 16


def _sc_body(idx_g, table, act, emb_out, cnt_out,
             idx_v, hidx_v, rows_v, ones_v, acc_v, act_v, hist_sh,
             sem0, sem1):
    c = lax.axis_index("c")
    s = lax.axis_index("s")
    wid = s * NC + c
    on_c0 = c == 0

    pltpu.sync_copy(idx_g.at[wid], idx_v)

    @pl.when(on_c0)
    def _stage_hist():
        pltpu.sync_copy(idx_g.at[2 * s], hidx_v.at[pl.ds(0, NCH)])
        pltpu.sync_copy(idx_g.at[2 * s + 1], hidx_v.at[pl.ds(NCH, NCH)])
        one = jnp.ones((LANES,), jnp.int32)
        zero = jnp.zeros((LANES,), jnp.int32)
        for i in range(CHUNK // LANES):
            ones_v[pl.ds(i * LANES, LANES)] = one
        for i in range(CSLICE // LANES):
            acc_v[pl.ds(i * LANES, LANES)] = zero
        pltpu.sync_copy(acc_v, hist_sh.at[pl.ds(s * CSLICE, CSLICE)])

    plsc.subcore_barrier()

    @pl.when(on_c0)
    def _histogram():
        def hstep(j, carry):
            pltpu.sync_copy(ones_v, hist_sh.at[hidx_v.at[j]], add=True)
            return carry
        lax.fori_loop(0, HCH, hstep, 0)

    def store_chunk(k, buf):
        b = wid * B_PER_W + k // CH_PER_B
        h0 = (k % CH_PER_B) * CHUNK
        pltpu.sync_copy(rows_v.at[buf], emb_out.at[b].at[pl.ds(h0, CHUNK)])

    def gpair(p, carry):
        k0 = p * 2
        d0 = pltpu.async_copy(table.at[idx_v.at[k0]], rows_v.at[0], sem0)
        d1 = pltpu.async_copy(table.at[idx_v.at[k0 + 1]], rows_v.at[1], sem1)
        d0.wait()
        store_chunk(k0, 0)
        d1.wait()
        store_chunk(k0 + 1, 1)
        return carry

    lax.fori_loop(0, NCH // 2, gpair, 0)

    plsc.subcore_barrier()

    @pl.when(on_c0)
    def _finalize_counts():
        sl = pl.ds(s * CSLICE, CSLICE)
        pltpu.sync_copy(hist_sh.at[sl], acc_v)
        pltpu.sync_copy(act.at[sl], act_v)
        for i in range(CSLICE // LANES):
            vsl = pl.ds(i * LANES, LANES)
            acc_v[vsl] = acc_v[vsl] + act_v[vsl]
        pltpu.sync_copy(acc_v, cnt_out.at[sl])


_sc_kernel = pl.kernel(
    _sc_body,
    out_type=(
        jax.ShapeDtypeStruct((BATCH, HW, DIM), jnp.float32),
        jax.ShapeDtypeStruct((NUM_PRIM,), jnp.int32),
    ),
    mesh=plsc.VectorSubcoreMesh(
        core_axis_name="c", subcore_axis_name="s",
        num_cores=NC, num_subcores=NS,
    ),
    compiler_params=pltpu.CompilerParams(use_tc_tiling_on_sc=False),
    scratch_types=[
        pltpu.VMEM((NCH, CHUNK), jnp.int32),
        pltpu.VMEM((HCH, CHUNK), jnp.int32),
        pltpu.VMEM((2, CHUNK, DIM), jnp.float32),
        pltpu.VMEM((CHUNK,), jnp.int32),
        pltpu.VMEM((CSLICE,), jnp.int32),
        pltpu.VMEM((CSLICE,), jnp.int32),
        pltpu.VMEM_SHARED((NUM_PRIM,), jnp.int32),
        pltpu.SemaphoreType.DMA,
        pltpu.SemaphoreType.DMA,
    ],
)

def _run(indices, primitives, activation_count):
    idx_g = indices.reshape(NW, NCH, CHUNK)
    return _sc_kernel(idx_g, primitives, activation_count)


_jitted = {}


def _get_jitted(dev):
    fn = _jitted.get(dev)
    if fn is None:
        sh = jax.sharding.SingleDeviceSharding(dev)
        fmts = (
            Format(Layout(major_to_minor=(0, 1, 2), tiling=((8,),)), sh),
            Format(None, sh),
        )
        fn = jax.jit(_run, out_shardings=fmts)
        _jitted[dev] = fn
    return fn


def kernel(indices, primitives, activation_count):
    dev = None
    try:
        dev = list(indices.devices())[0]
    except Exception:
        try:
            from jax._src import mesh as _mesh_lib
            cm = _mesh_lib.get_concrete_mesh()
            if cm is not None and cm.devices is not None:
                dev = cm.devices.flat[0]
        except Exception:
            dev = None
    if dev is None:
        dev = jax.devices()[0]
    return _get_jitted(dev)(indices, primitives, activation_count)
